# baseline (speedup 1.0000x reference)
import functools

import jax
import jax.numpy as jnp
from jax import lax
from jax.experimental import pallas as pl
from jax.experimental.pallas import tpu as pltpu
from jax.experimental.pallas import tpu_sc as plsc

def _lane_shuffle(v, idx):
    return lax.gather(
        v, idx[:, None],
        dimension_numbers=lax.GatherDimensionNumbers(
            offset_dims=(), collapsed_slice_dims=(0,), start_index_map=(0,)),
        slice_sizes=(1,),
        mode=lax.GatherScatterMode.PROMISE_IN_BOUNDS)


NC = 2
NS = 16
NW = NC * NS
LANES = 16


def _sc_kernel(x_flat, tbl_flat, B, L, D):
    RD = L * D
    ROWS_W = B // NW
    CH = 4
    NCHUNK = ROWS_W // CH
    TBL_W = tbl_flat.shape[0]

    mesh = plsc.VectorSubcoreMesh(core_axis_name="c", subcore_axis_name="s")

    @functools.partial(
        pl.kernel,
        mesh=mesh,
        out_type=jax.ShapeDtypeStruct((B * RD,), jnp.float32),
        scratch_types=[
            pltpu.VMEM((TBL_W,), jnp.float32),
            pltpu.VMEM((CH * RD,), jnp.float32),
            pltpu.VMEM((CH * RD,), jnp.float32),
            pltpu.SemaphoreType.DMA,
        ],
    )
    def k(x_hbm, tbl_hbm, out_hbm, tbl_v, ibuf, obuf, sem):
        wid = lax.axis_index("s") * NC + lax.axis_index("c")
        base_word = wid * ROWS_W * RD
        pltpu.async_copy(tbl_hbm, tbl_v, sem).wait()
        t0 = [tbl_v[pl.ds(16 * kk, LANES)] for kk in range(D // LANES)]

        def chunk_body(i, _):
            off = base_word + i * CH * RD
            pltpu.async_copy(x_hbm.at[pl.ds(off, CH * RD)], ibuf, sem).wait()

            def l_body(l, _):
                t1 = [tbl_v[pl.ds((l + 1) * D + 16 * kk, LANES)]
                      for kk in range(D // LANES)]
                for r in range(CH):
                    rb = r * RD + l * D
                    xs = [ibuf[pl.ds(rb + 16 * kk, LANES)]
                          for kk in range(D // LANES)]
                    s = jnp.abs(xs[0])
                    for v in xs[1:]:
                        s = s + jnp.abs(v)
                    for sh in (1, 2, 4, 8):
                        idx = jnp.arange(LANES, dtype=jnp.int32) ^ sh
                        s = s + _lane_shuffle(s, idx)
                    cond = s > 0.0
                    for kk in range(D // LANES):
                        obuf[pl.ds(rb + 16 * kk, LANES)] = (
                            xs[kk] + jnp.where(cond, t1[kk], t0[kk]))
                return 0

            lax.fori_loop(0, L, l_body, 0)
            pltpu.async_copy(obuf, out_hbm.at[pl.ds(off, CH * RD)], sem).wait()
            return 0

        lax.fori_loop(0, NCHUNK, chunk_body, 0)

    return k(x_flat, tbl_flat)


@jax.jit
def kernel(x, pos_table):
    B, L, D = x.shape
    out = _sc_kernel(x.reshape(-1), pos_table.reshape(-1), B, L, D)
    return out.reshape(B, L, D)


# speedup vs baseline: 1.0913x; 1.0913x over previous
import functools

import jax
import jax.numpy as jnp
from jax import lax
from jax.experimental import pallas as pl
from jax.experimental.pallas import tpu as pltpu
from jax.experimental.pallas import tpu_sc as plsc


def _lane_shuffle(v, idx):
    return lax.gather(
        v, idx[:, None],
        dimension_numbers=lax.GatherDimensionNumbers(
            offset_dims=(), collapsed_slice_dims=(0,), start_index_map=(0,)),
        slice_sizes=(1,),
        mode=lax.GatherScatterMode.PROMISE_IN_BOUNDS)


NC = 2
NS = 16
NW = NC * NS
LANES = 16


def _sc_kernel(x_flat, tbl_flat, B, L, D):
    RD = L * D
    ROWS_W = B // NW
    CH = 2
    NCHUNK = ROWS_W // CH
    CW = CH * RD
    TBL_W = tbl_flat.shape[0]

    mesh = plsc.VectorSubcoreMesh(core_axis_name="c", subcore_axis_name="s")

    @functools.partial(
        pl.kernel,
        mesh=mesh,
        out_type=jax.ShapeDtypeStruct((B * RD,), jnp.float32),
        scratch_types=[
            pltpu.VMEM((TBL_W,), jnp.float32),
            pltpu.VMEM((CW,), jnp.float32),
            pltpu.VMEM((CW,), jnp.float32),
            pltpu.VMEM((CW,), jnp.float32),
            pltpu.VMEM((CW,), jnp.float32),
            pltpu.SemaphoreType.DMA,
            pltpu.SemaphoreType.DMA,
            pltpu.SemaphoreType.DMA,
            pltpu.SemaphoreType.DMA,
            pltpu.SemaphoreType.DMA,
        ],
    )
    def k(x_hbm, tbl_hbm, out_hbm, tbl_v, ib0, ib1, ob0, ob1,
          tsem, is0, is1, os0, os1):
        wid = lax.axis_index("s") * NC + lax.axis_index("c")
        base = wid * ROWS_W * RD
        ibs, obs, iss, oss = (ib0, ib1), (ob0, ob1), (is0, is1), (os0, os1)
        pltpu.async_copy(tbl_hbm, tbl_v, tsem).wait()

        def cin(i, buf, sem):
            return pltpu.make_async_copy(
                x_hbm.at[pl.ds(base + i * CW, CW)], buf, sem)

        def cout(i, buf, sem):
            return pltpu.make_async_copy(
                buf, out_hbm.at[pl.ds(base + i * CW, CW)], sem)

        cin(0, ibs[0], iss[0]).start()
        cin(1, ibs[1], iss[1]).start()

        def step(g, _):
            for p in range(2):
                i = 2 * g + p
                ib, ob = ibs[p], obs[p]
                cin(i, ib, iss[p]).wait()

                @pl.when(i >= 2)
                def _():
                    cout(i - 2, ob, oss[p]).wait()

                _compute(ib, ob, tbl_v)
                cout(i, ob, oss[p]).start()

                @pl.when(i + 2 < NCHUNK)
                def _():
                    cin(i + 2, ib, iss[p]).start()
            return 0

        def _compute(ib, ob, tbl_v):
            def l_body(l, _):
                t1 = [tbl_v[pl.ds((l + 1) * D + 16 * kk, LANES)]
                      for kk in range(D // LANES)]
                t0 = [tbl_v[pl.ds(16 * kk, LANES)]
                      for kk in range(D // LANES)]
                for r in range(CH):
                    rb = r * RD + l * D
                    xs = [ib[pl.ds(rb + 16 * kk, LANES)]
                          for kk in range(D // LANES)]
                    s = jnp.abs(xs[0])
                    for v in xs[1:]:
                        s = s + jnp.abs(v)
                    for sh in (1, 2, 4, 8):
                        idx = jnp.arange(LANES, dtype=jnp.int32) ^ sh
                        s = s + _lane_shuffle(s, idx)
                    cond = s > 0.0
                    for kk in range(D // LANES):
                        ob[pl.ds(rb + 16 * kk, LANES)] = (
                            xs[kk] + jnp.where(cond, t1[kk], t0[kk]))
                return 0

            lax.fori_loop(0, L, l_body, 0)

        lax.fori_loop(0, NCHUNK // 2, step, 0)
        cout(NCHUNK - 2, obs[0], oss[0]).wait()
        cout(NCHUNK - 1, obs[1], oss[1]).wait()

    return k(x_flat, tbl_flat)


@jax.jit
def kernel(x, pos_table):
    B, L, D = x.shape
    out = _sc_kernel(x.reshape(-1), pos_table.reshape(-1), B, L, D)
    return out.reshape(B, L, D)


# speedup vs baseline: 3.2008x; 2.9332x over previous
import jax
import jax.numpy as jnp
from jax.experimental import pallas as pl


def _body(x_ref, tmain_ref, t0_ref, s_ref, o_ref):
    xb = x_ref[...]
    bb = xb.shape[0]
    f = (xb != 0.0).astype(jnp.float32)
    cnt = jax.lax.dot_general(
        f.reshape(bb * 100, 128), s_ref[...],
        (((1,), (0,)), ((), ())),
        preferred_element_type=jnp.float32,
    ).reshape(bb, 100, 128)
    emb = jnp.where(cnt > 0.0, tmain_ref[...][None], t0_ref[...][None])
    o_ref[...] = xb + emb


@jax.jit
def kernel(x, pos_table):
    B, L, D = x.shape
    BB = 128
    x2 = x.reshape(B, L // 2, 2 * D)
    tmain = pos_table[1:].reshape(L // 2, 2 * D)
    t0 = jnp.tile(pos_table[0], 2)[None, :]
    half = jnp.arange(2 * D, dtype=jnp.int32) // D
    s = (half[:, None] == half[None, :]).astype(jnp.float32)
    out = pl.pallas_call(
        _body,
        grid=(B // BB,),
        in_specs=[
            pl.BlockSpec((BB, L // 2, 2 * D), lambda i: (i, 0, 0)),
            pl.BlockSpec((L // 2, 2 * D), lambda i: (0, 0)),
            pl.BlockSpec((1, 2 * D), lambda i: (0, 0)),
            pl.BlockSpec((2 * D, 2 * D), lambda i: (0, 0)),
        ],
        out_specs=pl.BlockSpec((BB, L // 2, 2 * D), lambda i: (i, 0, 0)),
        out_shape=jax.ShapeDtypeStruct((B, L // 2, 2 * D), x.dtype),
    )(x2, tmain, t0, s)
    return out.reshape(B, L, D)
